# baseline (speedup 1.0000x reference)
import functools

import jax
import jax.numpy as jnp
from jax import lax
from jax.experimental import pallas as pl
from jax.experimental.pallas import tpu as pltpu
from jax.experimental.pallas import tpu_sc as plsc

SC_CORES = 2
SC_SUBCORES = 16
NW = SC_CORES * SC_SUBCORES
CHUNK = 64
NBUF = 2


def _sc_gather_sum(species_idx3, ability_idx3, item_idx3, move_idx3,
                   species_table, abilities_table, items_table, actions_table,
                   batch, dim):
  rows_per_w = batch // NW
  n_chunks = rows_per_w // CHUNK

  mesh = plsc.VectorSubcoreMesh(core_axis_name="c", subcore_axis_name="s")

  buf_set = [
      pltpu.VMEM((CHUNK, dim), jnp.float32),
      pltpu.VMEM((CHUNK, dim), jnp.float32),
      pltpu.VMEM((CHUNK, dim), jnp.float32),
      pltpu.VMEM((4 * CHUNK, dim), jnp.float32),
      pltpu.SemaphoreType.DMA,
      pltpu.SemaphoreType.DMA,
  ]

  @functools.partial(
      pl.kernel,
      out_type=jax.ShapeDtypeStruct((batch, dim), jnp.float32),
      mesh=mesh,
      scratch_types=[
          pltpu.VMEM((n_chunks, CHUNK), jnp.int32),
          pltpu.VMEM((n_chunks, CHUNK), jnp.int32),
          pltpu.VMEM((n_chunks, CHUNK), jnp.int32),
          pltpu.VMEM((4 * n_chunks, CHUNK), jnp.int32),
      ] + buf_set * NBUF,
  )
  def k(sp_hbm, ab_hbm, it_hbm, mv_hbm,
        sp_tbl, ab_tbl, it_tbl, ac_tbl, emb_hbm,
        sp_i, ab_i, it_i, mv_i, *bufs):
    wid = lax.axis_index("s") * SC_CORES + lax.axis_index("c")
    base = wid * rows_per_w
    pltpu.sync_copy(sp_hbm.at[wid], sp_i)
    pltpu.sync_copy(ab_hbm.at[wid], ab_i)
    pltpu.sync_copy(it_hbm.at[wid], it_i)
    pltpu.sync_copy(mv_hbm.at[wid], mv_i)

    sets = [bufs[6 * s:6 * (s + 1)] for s in range(NBUF)]
    wb = [None] * NBUF

    def fire(c, s):
      bsp, bab, bit, bmv, gsem, _ = sets[s]
      cps = [
          pltpu.async_copy(sp_tbl.at[sp_i.at[c]], bsp, gsem),
          pltpu.async_copy(ab_tbl.at[ab_i.at[c]], bab, gsem),
          pltpu.async_copy(it_tbl.at[it_i.at[c]], bit, gsem),
      ]
      for j in range(4):
        cps.append(pltpu.async_copy(
            ac_tbl.at[mv_i.at[4 * c + j]],
            bmv.at[pl.ds(j * CHUNK, CHUNK)], gsem))
      return cps

    inflight = [None] * NBUF
    inflight[0] = fire(0, 0)
    for c in range(n_chunks):
      s = c % NBUF
      bsp, bab, bit, bmv, gsem, wsem = sets[s]
      nxt = (c + 1) % NBUF
      if c + 1 < n_chunks:
        if wb[nxt] is not None:
          wb[nxt].wait()
          wb[nxt] = None
        inflight[nxt] = fire(c + 1, nxt)
      for cp in inflight[s]:
        cp.wait()
      inflight[s] = None

      def row_sum(r, _):
        for l in range(dim // 16):
          lane = pl.ds(l * 16, 16)
          v = bsp[r, lane] + bab[r, lane] + bit[r, lane]
          v = v + bmv[4 * r, lane] + bmv[4 * r + 1, lane]
          v = v + bmv[4 * r + 2, lane] + bmv[4 * r + 3, lane]
          bsp[r, lane] = v
        return 0
      lax.fori_loop(0, CHUNK, row_sum, 0)

      wb[s] = pltpu.async_copy(
          bsp, emb_hbm.at[pl.ds(base + c * CHUNK, CHUNK)], wsem)
    for s in range(NBUF):
      if wb[s] is not None:
        wb[s].wait()

  return k(species_idx3, ability_idx3, item_idx3, move_idx3,
           species_table, abilities_table, items_table, actions_table)


def _tc_mlp_body(emb_ref, w_ref, b_ref, sidx_ref, out_ref):
  h = jnp.dot(emb_ref[...], w_ref[...], preferred_element_type=jnp.float32)
  h = jnp.maximum(h + b_ref[...], 0.0)
  s = sidx_ref[...]
  mask = jnp.logical_and(s != 0, s != 1)
  out_ref[...] = jnp.where(mask, h, 0.0)


N_SLICES = 2


def kernel(species_idx, ability_idx, item_idx, move_idx,
           species_table, abilities_table, items_table, actions_table,
           W1, b1):
  batch = species_idx.shape[0]
  dim = W1.shape[0]
  sb = batch // N_SLICES
  rows_per_w = sb // NW
  n_chunks = rows_per_w // CHUNK
  b1r = b1.reshape(1, dim)

  rows = 1024
  outs = []
  for s in range(N_SLICES):
    sl = slice(s * sb, (s + 1) * sb)
    emb = _sc_gather_sum(
        species_idx[sl].reshape(NW, n_chunks, CHUNK),
        ability_idx[sl].reshape(NW, n_chunks, CHUNK),
        item_idx[sl].reshape(NW, n_chunks, CHUNK),
        move_idx[sl].reshape(NW, 4 * n_chunks, CHUNK),
        species_table, abilities_table, items_table, actions_table,
        sb, dim)
    outs.append(pl.pallas_call(
        _tc_mlp_body,
        grid=(sb // rows,),
        in_specs=[
            pl.BlockSpec((rows, dim), lambda i: (i, 0)),
            pl.BlockSpec((dim, dim), lambda i: (0, 0)),
            pl.BlockSpec((1, dim), lambda i: (0, 0)),
            pl.BlockSpec((rows, 1), lambda i: (i, 0)),
        ],
        out_specs=pl.BlockSpec((rows, dim), lambda i: (i, 0)),
        out_shape=jax.ShapeDtypeStruct((sb, dim), jnp.float32),
    )(emb, W1, b1r, species_idx[sl].reshape(sb, 1)))
  return jnp.concatenate(outs, axis=0)


# speedup vs baseline: 1.0635x; 1.0635x over previous
import functools

import jax
import jax.numpy as jnp
from jax import lax
from jax.experimental import pallas as pl
from jax.experimental.pallas import tpu as pltpu
from jax.experimental.pallas import tpu_sc as plsc

SC_CORES = 2
SC_SUBCORES = 16
NW = SC_CORES * SC_SUBCORES
CHUNK = 64
NBUF = 2


def _sc_gather_sum(species_idx3, ability_idx3, item_idx3, move_idx3,
                   species_table, abilities_table, items_table, actions_table,
                   batch, dim):
  rows_per_w = batch // NW
  n_chunks = rows_per_w // CHUNK

  mesh = plsc.VectorSubcoreMesh(core_axis_name="c", subcore_axis_name="s")

  buf_set = [
      pltpu.VMEM((CHUNK, dim), jnp.float32),
      pltpu.VMEM((CHUNK, dim), jnp.float32),
      pltpu.VMEM((CHUNK, dim), jnp.float32),
      pltpu.VMEM((4 * CHUNK, dim), jnp.float32),
      pltpu.SemaphoreType.DMA,
      pltpu.SemaphoreType.DMA,
  ]

  @functools.partial(
      pl.kernel,
      out_type=jax.ShapeDtypeStruct((batch, dim), jnp.float32),
      mesh=mesh,
      scratch_types=[
          pltpu.VMEM((n_chunks, CHUNK), jnp.int32),
          pltpu.VMEM((n_chunks, CHUNK), jnp.int32),
          pltpu.VMEM((n_chunks, CHUNK), jnp.int32),
          pltpu.VMEM((4 * n_chunks, CHUNK), jnp.int32),
      ] + buf_set * NBUF,
  )
  def k(sp_hbm, ab_hbm, it_hbm, mv_hbm,
        sp_tbl, ab_tbl, it_tbl, ac_tbl, emb_hbm,
        sp_i, ab_i, it_i, mv_i, *bufs):
    wid = lax.axis_index("s") * SC_CORES + lax.axis_index("c")
    base = wid * rows_per_w
    pltpu.sync_copy(sp_hbm.at[wid], sp_i)
    pltpu.sync_copy(ab_hbm.at[wid], ab_i)
    pltpu.sync_copy(it_hbm.at[wid], it_i)
    pltpu.sync_copy(mv_hbm.at[wid], mv_i)

    sets = [bufs[6 * s:6 * (s + 1)] for s in range(NBUF)]
    wb = [None] * NBUF

    def fire(c, s):
      bsp, bab, bit, bmv, gsem, _ = sets[s]
      cps = [
          pltpu.async_copy(sp_tbl.at[sp_i.at[c]], bsp, gsem),
          pltpu.async_copy(ab_tbl.at[ab_i.at[c]], bab, gsem),
          pltpu.async_copy(it_tbl.at[it_i.at[c]], bit, gsem),
      ]
      for j in range(4):
        cps.append(pltpu.async_copy(
            ac_tbl.at[mv_i.at[4 * c + j]],
            bmv.at[pl.ds(j * CHUNK, CHUNK)], gsem))
      return cps

    inflight = [None] * NBUF
    inflight[0] = fire(0, 0)
    for c in range(n_chunks):
      s = c % NBUF
      bsp, bab, bit, bmv, gsem, wsem = sets[s]
      nxt = (c + 1) % NBUF
      if c + 1 < n_chunks:
        if wb[nxt] is not None:
          wb[nxt].wait()
          wb[nxt] = None
        inflight[nxt] = fire(c + 1, nxt)
      for cp in inflight[s]:
        cp.wait()
      inflight[s] = None

      def row_sum(r, _):
        for l in range(dim // 16):
          lane = pl.ds(l * 16, 16)
          v = bsp[r, lane] + bab[r, lane] + bit[r, lane]
          v = v + bmv[4 * r, lane] + bmv[4 * r + 1, lane]
          v = v + bmv[4 * r + 2, lane] + bmv[4 * r + 3, lane]
          bsp[r, lane] = v
        return 0
      lax.fori_loop(0, CHUNK, row_sum, 0)

      wb[s] = pltpu.async_copy(
          bsp, emb_hbm.at[pl.ds(base + c * CHUNK, CHUNK)], wsem)
    for s in range(NBUF):
      if wb[s] is not None:
        wb[s].wait()

  return k(species_idx3, ability_idx3, item_idx3, move_idx3,
           species_table, abilities_table, items_table, actions_table)


def _tc_mlp_body(emb_ref, w_ref, b_ref, sidx_ref, out_ref):
  h = jnp.dot(emb_ref[...], w_ref[...], preferred_element_type=jnp.float32)
  h = jnp.maximum(h + b_ref[...], 0.0)
  s = sidx_ref[...]
  mask = jnp.logical_and(s != 0, s != 1)
  out_ref[...] = jnp.where(mask, h, 0.0)


N_SLICES = 1


def kernel(species_idx, ability_idx, item_idx, move_idx,
           species_table, abilities_table, items_table, actions_table,
           W1, b1):
  batch = species_idx.shape[0]
  dim = W1.shape[0]
  sb = batch // N_SLICES
  rows_per_w = sb // NW
  n_chunks = rows_per_w // CHUNK
  b1r = b1.reshape(1, dim)

  rows = 1024
  outs = []
  for s in range(N_SLICES):
    sl = slice(s * sb, (s + 1) * sb)
    emb = _sc_gather_sum(
        species_idx[sl].reshape(NW, n_chunks, CHUNK),
        ability_idx[sl].reshape(NW, n_chunks, CHUNK),
        item_idx[sl].reshape(NW, n_chunks, CHUNK),
        move_idx[sl].reshape(NW, 4 * n_chunks, CHUNK),
        species_table, abilities_table, items_table, actions_table,
        sb, dim)
    outs.append(pl.pallas_call(
        _tc_mlp_body,
        grid=(sb // rows,),
        in_specs=[
            pl.BlockSpec((rows, dim), lambda i: (i, 0)),
            pl.BlockSpec((dim, dim), lambda i: (0, 0)),
            pl.BlockSpec((1, dim), lambda i: (0, 0)),
            pl.BlockSpec((rows, 1), lambda i: (i, 0)),
        ],
        out_specs=pl.BlockSpec((rows, dim), lambda i: (i, 0)),
        out_shape=jax.ShapeDtypeStruct((sb, dim), jnp.float32),
    )(emb, W1, b1r, species_idx[sl].reshape(sb, 1)))
  return jnp.concatenate(outs, axis=0)
